# baseline (speedup 1.0000x reference)
import functools

import jax
import jax.numpy as jnp
from jax import lax
from jax.experimental import pallas as pl
from jax.experimental.pallas import tpu as pltpu
from jax.experimental.pallas import tpu_sc as plsc

EMBED_DIM = 64
LANES = 16
CHUNK = 512
IDX_MINOR = 128
GATHERS = CHUNK // IDX_MINOR
COL_CHUNKS = EMBED_DIM // LANES
TEST_UNROLL = 8
FIX_UNROLL = 4


def _make_sc_gather(n_rows: int):
  info = plsc.get_sparse_core_info()
  nc, ns = info.num_cores, info.num_subcores
  nw = nc * ns
  assert n_rows % (nw * CHUNK) == 0
  chunks_per_w = n_rows // (nw * CHUNK)
  rows_per_w = chunks_per_w * CHUNK
  idx_rows_per_w = rows_per_w // IDX_MINOR

  mesh = plsc.VectorSubcoreMesh(core_axis_name="c", subcore_axis_name="s")

  @functools.partial(
      pl.kernel,
      out_type=jax.ShapeDtypeStruct((n_rows, 2 * EMBED_DIM), jnp.float32),
      mesh=mesh,
      scratch_types=[
          pltpu.VMEM((idx_rows_per_w, IDX_MINOR), jnp.int32),
          pltpu.VMEM((CHUNK, EMBED_DIM), jnp.float32),
          pltpu.VMEM((CHUNK, EMBED_DIM), jnp.float32),
          pltpu.VMEM((EMBED_DIM,), jnp.float32),
          pltpu.SemaphoreType.DMA,
          pltpu.SemaphoreType.DMA,
          pltpu.SemaphoreType.DMA,
          pltpu.SemaphoreType.DMA,
      ],
      compiler_params=pltpu.CompilerParams(
          use_tc_tiling_on_sc=False, needs_layout_passes=False),
  )
  def sc_kernel(x_hbm, table_hbm, out_hbm, idx_v, buf0, buf1, zv_v,
                gsem0, gsem1, wsem0, wsem1):
    wid = lax.axis_index("s") * nc + lax.axis_index("c")
    row_base = wid * rows_per_w
    pltpu.sync_copy(table_hbm.at[0], zv_v)
    pltpu.sync_copy(x_hbm.at[pl.ds(wid * idx_rows_per_w, idx_rows_per_w)],
                    idx_v)
    zvc = [zv_v[pl.ds(c * LANES, LANES)] for c in range(COL_CHUNKS)]

    bufs = (buf0, buf1)
    gsems = (gsem0, gsem1)
    wsems = (wsem0, wsem1)

    def fire_gather(g, buf, gsem):
      for j in range(GATHERS):
        pltpu.async_copy(
            table_hbm.at[idx_v.at[g * GATHERS + j]],
            buf.at[pl.ds(j * IDX_MINOR, IDX_MINOR)],
            gsem,
        )

    def drain_gather(g, buf, gsem):
      for j in range(GATHERS):
        pltpu.make_async_copy(
            table_hbm.at[idx_v.at[g * GATHERS + j]],
            buf.at[pl.ds(j * IDX_MINOR, IDX_MINOR)],
            gsem,
        ).wait()

    def fire_wb(g, buf, wsem):
      pltpu.async_copy(
          buf,
          out_hbm.at[pl.ds(row_base + g * CHUNK, CHUNK), pl.ds(0, EMBED_DIM)],
          wsem)

    def drain_wb(g, buf, wsem):
      pltpu.make_async_copy(
          buf,
          out_hbm.at[pl.ds(row_base + g * CHUNK, CHUNK), pl.ds(0, EMBED_DIM)],
          wsem).wait()

    def fixup(buf):
      def row_body(i, carry2):
        for u in range(FIX_UNROLL):
          r = i * FIX_UNROLL + u
          vals = [buf[r, pl.ds(c * LANES, LANES)] for c in range(COL_CHUNKS)]
          d = vals[0] != zvc[0]
          for c in range(1, COL_CHUNKS):
            d = jnp.logical_or(d, vals[c] != zvc[c])
          nf = d.astype(jnp.float32)
          for shift in (8, 4, 2, 1):
            perm = (lax.iota(jnp.int32, LANES) + shift) % LANES
            nf = jnp.maximum(nf, nf.at[perm].get(mode="promise_in_bounds"))
          for c in range(COL_CHUNKS):
            buf[r, pl.ds(c * LANES, LANES)] = vals[c] * nf
        return carry2

      lax.fori_loop(0, CHUNK // FIX_UNROLL, row_body, 0)

    def process(buf):
      def test_body(i, acc):
        for u in range(TEST_UNROLL):
          r = i * TEST_UNROLL + u
          acc = jnp.logical_or(acc, buf[r, pl.ds(0, LANES)] == zvc[0])
        return acc

      acc0 = jnp.zeros((LANES,), jnp.bool_)
      acc = lax.fori_loop(0, CHUNK // TEST_UNROLL, test_body, acc0)
      maybe = jnp.any(acc)
      pl.when(maybe)(lambda: fixup(buf))

    fire_gather(0, buf0, gsem0)

    def chunk_pair(i, carry):
      for p in range(2):
        g = i * 2 + p
        buf, gsem, wsem = bufs[p], gsems[p], wsems[p]
        nbuf, ngsem, nwsem = bufs[1 - p], gsems[1 - p], wsems[1 - p]
        drain_gather(g, buf, gsem)

        @pl.when(g + 1 < chunks_per_w)
        def _():
          @pl.when(g >= 1)
          def _():
            drain_wb(g - 1, nbuf, nwsem)
          fire_gather(g + 1, nbuf, ngsem)

        process(buf)
        fire_wb(g, buf, wsem)
      return carry

    lax.fori_loop(0, chunks_per_w // 2, chunk_pair, 0)
    drain_wb(chunks_per_w - 2, buf0, wsem0)
    drain_wb(chunks_per_w - 1, buf1, wsem1)

  return sc_kernel


def _make_tc_pack(v: int):
  bw = 4000

  def body(in_ref, out_ref):
    blk = in_ref[...].reshape(bw // 2, 2, EMBED_DIM)
    out_ref[...] = jnp.concatenate([blk[:, 0, :], blk[:, 1, :]], axis=1)

  return pl.pallas_call(
      body,
      out_shape=jax.ShapeDtypeStruct((v // 2, 2 * EMBED_DIM), jnp.float32),
      grid=(v // bw,),
      in_specs=[pl.BlockSpec((bw, EMBED_DIM), lambda i: (i, 0))],
      out_specs=pl.BlockSpec((bw // 2, 2 * EMBED_DIM), lambda i: (i, 0)),
  )


def kernel(x, table):
  b, l = x.shape
  n = b * l
  v = table.shape[0]
  table_lin = _make_tc_pack(v)(table).reshape(v, EMBED_DIM)
  x2 = x.reshape(n // IDX_MINOR, IDX_MINOR)
  out = _make_sc_gather(n)(x2, table_lin)
  return out.reshape(b, l, 2 * EMBED_DIM)[:, :, :EMBED_DIM]


# speedup vs baseline: 1.6447x; 1.6447x over previous
import functools

import jax
import jax.numpy as jnp
from jax import lax
from jax.experimental import pallas as pl
from jax.experimental.pallas import tpu as pltpu
from jax.experimental.pallas import tpu_sc as plsc
from jax.experimental import layout as jexp_layout

EMBED_DIM = 64
LANES = 16
CHUNK = 512
IDX_MINOR = 128
GATHERS = CHUNK // IDX_MINOR
COL_CHUNKS = EMBED_DIM // LANES
TEST_UNROLL = 8
FIX_UNROLL = 4


def _make_sc_gather(n_rows: int):
  info = plsc.get_sparse_core_info()
  nc, ns = info.num_cores, info.num_subcores
  nw = nc * ns
  assert n_rows % (nw * CHUNK) == 0
  chunks_per_w = n_rows // (nw * CHUNK)
  rows_per_w = chunks_per_w * CHUNK
  idx_rows_per_w = rows_per_w // IDX_MINOR

  mesh = plsc.VectorSubcoreMesh(core_axis_name="c", subcore_axis_name="s")

  @functools.partial(
      pl.kernel,
      out_type=jax.ShapeDtypeStruct((n_rows, 2 * EMBED_DIM), jnp.float32),
      name="sc_gather_mask",
      mesh=mesh,
      scratch_types=[
          pltpu.VMEM((idx_rows_per_w, IDX_MINOR), jnp.int32),
          pltpu.VMEM((CHUNK, EMBED_DIM), jnp.float32),
          pltpu.VMEM((CHUNK, EMBED_DIM), jnp.float32),
          pltpu.VMEM((EMBED_DIM,), jnp.float32),
          pltpu.SemaphoreType.DMA,
          pltpu.SemaphoreType.DMA,
          pltpu.SemaphoreType.DMA,
          pltpu.SemaphoreType.DMA,
      ],
      compiler_params=pltpu.CompilerParams(
          use_tc_tiling_on_sc=False, needs_layout_passes=False),
  )
  def sc_kernel(x_hbm, table_hbm, out_hbm, idx_v, buf0, buf1, zv_v,
                gsem0, gsem1, wsem0, wsem1):
    wid = lax.axis_index("s") * nc + lax.axis_index("c")
    row_base = wid * rows_per_w
    pltpu.sync_copy(table_hbm.at[0], zv_v)
    pltpu.sync_copy(x_hbm.at[pl.ds(wid * idx_rows_per_w, idx_rows_per_w)],
                    idx_v)
    zvc = [zv_v[pl.ds(c * LANES, LANES)] for c in range(COL_CHUNKS)]

    bufs = (buf0, buf1)
    gsems = (gsem0, gsem1)
    wsems = (wsem0, wsem1)

    def fire_gather(g, buf, gsem):
      for j in range(GATHERS):
        pltpu.async_copy(
            table_hbm.at[idx_v.at[g * GATHERS + j]],
            buf.at[pl.ds(j * IDX_MINOR, IDX_MINOR)],
            gsem,
        )

    def drain_gather(g, buf, gsem):
      for j in range(GATHERS):
        pltpu.make_async_copy(
            table_hbm.at[idx_v.at[g * GATHERS + j]],
            buf.at[pl.ds(j * IDX_MINOR, IDX_MINOR)],
            gsem,
        ).wait()

    def fire_wb(g, buf, wsem):
      pltpu.async_copy(
          buf,
          out_hbm.at[pl.ds(row_base + g * CHUNK, CHUNK), pl.ds(0, EMBED_DIM)],
          wsem)

    def drain_wb(g, buf, wsem):
      pltpu.make_async_copy(
          buf,
          out_hbm.at[pl.ds(row_base + g * CHUNK, CHUNK), pl.ds(0, EMBED_DIM)],
          wsem).wait()

    def fixup(buf):
      def row_body(i, carry2):
        for u in range(FIX_UNROLL):
          r = i * FIX_UNROLL + u
          vals = [buf[r, pl.ds(c * LANES, LANES)] for c in range(COL_CHUNKS)]
          d = vals[0] != zvc[0]
          for c in range(1, COL_CHUNKS):
            d = jnp.logical_or(d, vals[c] != zvc[c])
          nf = d.astype(jnp.float32)
          for shift in (8, 4, 2, 1):
            perm = (lax.iota(jnp.int32, LANES) + shift) % LANES
            nf = jnp.maximum(nf, nf.at[perm].get(mode="promise_in_bounds"))
          for c in range(COL_CHUNKS):
            buf[r, pl.ds(c * LANES, LANES)] = vals[c] * nf
        return carry2

      lax.fori_loop(0, CHUNK // FIX_UNROLL, row_body, 0)

    def process(buf):
      def test_body(i, acc):
        for u in range(TEST_UNROLL):
          r = i * TEST_UNROLL + u
          acc = jnp.logical_or(acc, buf[r, pl.ds(0, LANES)] == zvc[0])
        return acc

      acc0 = jnp.zeros((LANES,), jnp.bool_)
      acc = lax.fori_loop(0, CHUNK // TEST_UNROLL, test_body, acc0)
      maybe = jnp.any(acc)
      pl.when(maybe)(lambda: fixup(buf))

    fire_gather(0, buf0, gsem0)

    def chunk_pair(i, carry):
      for p in range(2):
        g = i * 2 + p
        buf, gsem, wsem = bufs[p], gsems[p], wsems[p]
        nbuf, ngsem, nwsem = bufs[1 - p], gsems[1 - p], wsems[1 - p]
        drain_gather(g, buf, gsem)

        @pl.when(g + 1 < chunks_per_w)
        def _():
          @pl.when(g >= 1)
          def _():
            drain_wb(g - 1, nbuf, nwsem)
          fire_gather(g + 1, nbuf, ngsem)

        process(buf)
        fire_wb(g, buf, wsem)
      return carry

    lax.fori_loop(0, chunks_per_w // 2, chunk_pair, 0)
    drain_wb(chunks_per_w - 2, buf0, wsem0)
    drain_wb(chunks_per_w - 1, buf1, wsem1)

  return sc_kernel


def kernel(x, table):
  b, l = x.shape
  n = b * l
  table_lin = jexp_layout.with_layout_constraint(
      table, jexp_layout.Layout((0, 1), tiling=()))
  x2 = x.reshape(n // IDX_MINOR, IDX_MINOR)
  out = _make_sc_gather(n)(x2, table_lin)
  return out.reshape(b, l, 2 * EMBED_DIM)[:, :, :EMBED_DIM]
